# baseline (speedup 1.0000x reference)
import functools

import jax
import jax.numpy as jnp
from jax import lax
from jax.experimental import pallas as pl
from jax.experimental.pallas import tpu as pltpu

_BETA = 6.6

_P0 = 72
_W = 66
_M = 64 * _W
_L = 2 * _P0 + _M

_P0b = 136
_Wb = 130
_Mb = 128 * _Wb
_Lb = 16912

_S2MAP = {0: (1, -1), 1: (0, 0), 2: (1, 0), 3: (0, 1)}
_T1MAP = {0: ((0, 1), (-1, 3)), 1: ((1, 0), (0, 2))}

_OFF9 = [(ao, bo) for ao in (-1, 0, 1) for bo in (-1, 0, 1)]


def _cparams():
    return pltpu.CompilerParams(
        dimension_semantics=("parallel",),
        vmem_limit_bytes=60 * 1024 * 1024,
    )


def _dot(a, b):
    return jnp.dot(a, b, preferred_element_type=jnp.float32)


def _valid_mask():
    return (lax.broadcasted_iota(jnp.int32, (_M, 1), 0) % _W) < 64


def _store_padded(ref, val, c):
    ref[0:_P0, :] = jnp.zeros((_P0, c), jnp.float32)
    ref[_P0:_P0 + _M, :] = val
    ref[_P0 + _M:_L, :] = jnp.zeros((_L - _P0 - _M, c), jnp.float32)


def _conv3x3_acc(src, wtaps_ref, relu_src):
    acc = None
    for t, (ao, bo) in enumerate(_OFF9):
        st = _P0 + _W * ao + bo
        x = src[st:st + _M, :]
        if relu_src:
            x = jnp.maximum(x, 0.0)
        c = _dot(x, wtaps_ref[t])
        acc = c if acc is None else acc + c
    return acc


def _k_conv1(pat_ref, w_ref, b_ref, o_ref):
    y = _dot(pat_ref[0], w_ref[...]) + b_ref[...]
    o_ref[0] = jnp.maximum(y, 0.0)


def _k_conv2(in_ref, w_ref, b_ref, o_ref):
    acc = None
    for dy in range(4):
        r, ao = _S2MAP[dy]
        for dx in range(4):
            s, bo = _S2MAP[dx]
            p = r * 2 + s
            st = _P0 + _W * ao + bo
            c = _dot(in_ref[0, p, st:st + _M, :], w_ref[dy * 4 + dx])
            acc = c if acc is None else acc + c
    z = jnp.maximum(acc + b_ref[...], 0.0)
    z = jnp.where(_valid_mask(), z, 0.0)
    _store_padded(o_ref.at[0], z, 128)


def _k_enc_trunk(in_ref, w3_ref, b3_ref, r0a_ref, r0b_ref, r1a_ref, r1b_ref,
                 wp_ref, bp_ref, o_ref, sz_ref, sa_ref):
    valid = _valid_mask()
    z3 = _conv3x3_acc(in_ref.at[0], w3_ref, False) + b3_ref[...]
    _store_padded(sz_ref, jnp.where(valid, z3, 0.0), 128)
    for ra, rb in ((r0a_ref, r0b_ref), (r1a_ref, r1b_ref)):
        ta = _conv3x3_acc(sz_ref, ra, True)
        _store_padded(sa_ref, jnp.where(valid, jnp.maximum(ta, 0.0), 0.0), 32)
        tb = _conv3x3_acc(sa_ref, rb, False)
        sz_ref[_P0:_P0 + _M, :] = (sz_ref[_P0:_P0 + _M, :]
                                   + jnp.where(valid, tb, 0.0))
    h = jnp.maximum(sz_ref[_P0:_P0 + _M, :], 0.0)
    o_ref[0] = _dot(h, wp_ref[...]) + bp_ref[...]


def _k_vq(tok_ref, et_ref, e_ref, q_ref, cnt_ref, se_ref):
    valid = _valid_mask()
    z = tok_ref[0]
    zn = jnp.sum(z * z, axis=1, keepdims=True)
    e2 = jnp.sum(et_ref[...] * et_ref[...], axis=0, keepdims=True)
    d = (zn + e2) - 2.0 * _dot(z, et_ref[...])
    idx = jnp.argmin(d, axis=1)
    oh = jnp.where(
        lax.broadcasted_iota(jnp.int32, (_M, 512), 1) == idx[:, None],
        1.0, 0.0)
    q = _dot(oh, e_ref[...])
    vf = jnp.where(valid, 1.0, 0.0)
    qm = q * vf
    q_ref[0] = qm
    cnt_ref[0] = jnp.broadcast_to(jnp.sum(oh * vf, axis=0, keepdims=True),
                                  (8, 512))
    se = jnp.sum((qm - z * vf) ** 2)
    se_ref[0] = jnp.full((8, 128), se, jnp.float32)


def _k_dec_trunk(q_ref, w1_ref, b1_ref, r0a_ref, r0b_ref, r1a_ref, r1b_ref,
                 o_ref, si_ref, sz_ref, sa_ref):
    valid = _valid_mask()
    _store_padded(si_ref, q_ref[0], 64)
    d1 = _conv3x3_acc(si_ref, w1_ref, False) + b1_ref[...]
    _store_padded(sz_ref, jnp.where(valid, d1, 0.0), 128)
    for ra, rb in ((r0a_ref, r0b_ref), (r1a_ref, r1b_ref)):
        ta = _conv3x3_acc(sz_ref, ra, True)
        _store_padded(sa_ref, jnp.where(valid, jnp.maximum(ta, 0.0), 0.0), 32)
        tb = _conv3x3_acc(sa_ref, rb, False)
        sz_ref[_P0:_P0 + _M, :] = (sz_ref[_P0:_P0 + _M, :]
                                   + jnp.where(valid, tb, 0.0))
    h = jnp.maximum(sz_ref[_P0:_P0 + _M, :], 0.0)
    _store_padded(o_ref.at[0], h, 128)


def _k_convt1(in_ref, w_ref, b_ref, o_ref):
    for r in (0, 1):
        for s in (0, 1):
            acc = None
            for ao, dy in _T1MAP[r]:
                for bo, dx in _T1MAP[s]:
                    st = _P0 + _W * ao + bo
                    c = _dot(in_ref[0, st:st + _M, :], w_ref[dy * 4 + dx])
                    acc = c if acc is None else acc + c
            o_ref[0, r * 2 + s] = jnp.maximum(acc + b_ref[...], 0.0)


def _k_convt2(in_ref, w_ref, b_ref, o_ref):
    chunk = _Mb // 4
    for k in range(4):
        acc = None
        for t, (ao, bo) in enumerate(_OFF9):
            st = _P0b + _Wb * ao + bo + k * chunk
            c = _dot(in_ref[0, st:st + chunk, :], w_ref[t])
            acc = c if acc is None else acc + c
        o_ref[0, k * chunk:(k + 1) * chunk, :] = acc + b_ref[...]


def _bspec(shape, batched):
    if batched:
        return pl.BlockSpec((1,) + shape[1:],
                            lambda i: (i,) + (0,) * (len(shape) - 1))
    return pl.BlockSpec(shape, lambda i: (0,) * len(shape))


def _call(body, batch, ins, batched_flags, out_shapes, batched_out, scratch=()):
    in_specs = [_bspec(a.shape, f) for a, f in zip(ins, batched_flags)]
    out_specs = jax.tree.map(
        lambda s, f: _bspec(s.shape, f), out_shapes, batched_out)
    return pl.pallas_call(
        body,
        grid=(batch,),
        in_specs=in_specs,
        out_specs=out_specs,
        out_shape=out_shapes,
        scratch_shapes=list(scratch),
        compiler_params=_cparams(),
    )(*ins)


def kernel(x, w1, b1, w2, b2, w3, b3, r0a, r0b, r1a, r1b, wp, bp, E,
           dw1, db1, dr0a, dr0b, dr1a, dr1b, tw1, tb1, tw2, tb2):
    B = x.shape[0]
    f32 = jnp.float32

    xpad = jnp.pad(x, ((0, 0), (0, 0), (1, 1), (1, 1)))
    taps = [xpad[:, :, dy:dy + 256:2, dx:dx + 256:2]
            for dy in range(4) for dx in range(4)]
    pat = jnp.stack(taps, axis=2)
    pat = pat.transpose(0, 3, 4, 2, 1).reshape(B, 16384, 48)
    w1m = w1.transpose(2, 3, 1, 0).reshape(48, 64)
    y1 = _call(_k_conv1, B, (pat, w1m, b1), (True, False, False),
               jax.ShapeDtypeStruct((B, 16384, 64), f32), True)

    y1i = y1.reshape(B, 128, 128, 64)
    phases = [y1i[:, r::2, s::2, :] for r in (0, 1) for s in (0, 1)]
    ph = jnp.stack(phases, axis=1)
    ph = jnp.pad(ph, ((0, 0), (0, 0), (0, 0), (0, 2), (0, 0)))
    ph = ph.reshape(B, 4, _M, 64)
    ph = jnp.pad(ph, ((0, 0), (0, 0), (_P0, _L - _P0 - _M), (0, 0)))
    w2m = w2.transpose(2, 3, 1, 0).reshape(16, 64, 128)
    z2 = _call(_k_conv2, B, (ph, w2m, b2), (True, False, False),
               jax.ShapeDtypeStruct((B, _L, 128), f32), True)

    w3m = w3.transpose(2, 3, 1, 0).reshape(9, 128, 128)
    r0am = r0a.transpose(2, 3, 1, 0).reshape(9, 128, 32)
    r0bm = r0b.transpose(2, 3, 1, 0).reshape(9, 32, 128)
    r1am = r1a.transpose(2, 3, 1, 0).reshape(9, 128, 32)
    r1bm = r1b.transpose(2, 3, 1, 0).reshape(9, 32, 128)
    wpm = wp[:, :, 0, 0].T
    tok = _call(_k_enc_trunk, B,
                (z2, w3m, b3, r0am, r0bm, r1am, r1bm, wpm, bp),
                (True,) + (False,) * 8,
                jax.ShapeDtypeStruct((B, _M, 64), f32), True,
                scratch=(pltpu.VMEM((_L, 128), f32),
                         pltpu.VMEM((_L, 32), f32)))

    Et = E.T
    q, cnt, se = _call(
        _k_vq, B, (tok, Et, E), (True, False, False),
        (jax.ShapeDtypeStruct((B, _M, 64), f32),
         jax.ShapeDtypeStruct((B, 8, 512), f32),
         jax.ShapeDtypeStruct((B, 8, 128), f32)),
        (True, True, True))
    n_tok = B * 4096
    counts = jnp.sum(cnt[:, 0, :], axis=0)
    probs = counts / n_tok
    perplexity = jnp.exp(-jnp.sum(probs * jnp.log(probs + 1e-10)))
    loss = (1.0 + _BETA) * jnp.sum(se[:, 0, 0]) / (n_tok * 64)

    dw1m = dw1.transpose(2, 3, 1, 0).reshape(9, 64, 128)
    dr0am = dr0a.transpose(2, 3, 1, 0).reshape(9, 128, 32)
    dr0bm = dr0b.transpose(2, 3, 1, 0).reshape(9, 32, 128)
    dr1am = dr1a.transpose(2, 3, 1, 0).reshape(9, 128, 32)
    dr1bm = dr1b.transpose(2, 3, 1, 0).reshape(9, 32, 128)
    dtr = _call(_k_dec_trunk, B,
                (q, dw1m, db1, dr0am, dr0bm, dr1am, dr1bm),
                (True,) + (False,) * 6,
                jax.ShapeDtypeStruct((B, _L, 128), f32), True,
                scratch=(pltpu.VMEM((_L, 64), f32),
                         pltpu.VMEM((_L, 128), f32),
                         pltpu.VMEM((_L, 32), f32)))

    tw1m = tw1.transpose(2, 3, 0, 1).reshape(16, 128, 64)
    u = _call(_k_convt1, B, (dtr, tw1m, tb1), (True, False, False),
              jax.ShapeDtypeStruct((B, 4, _M, 64), f32), True)

    u = u.reshape(B, 2, 2, 64, _W, 64)[:, :, :, :, :64, :]
    u = u.transpose(0, 3, 1, 4, 2, 5).reshape(B, 128, 128, 64)
    u = jnp.pad(u, ((0, 0), (0, 0), (1, 1), (0, 0)))
    u = u.reshape(B, _Mb, 64)
    u = jnp.pad(u, ((0, 0), (_P0b, _Lb - _P0b - _Mb), (0, 0)))

    w6 = jnp.zeros((3, 3, 2, 2, 64, 3), f32)
    for r in (0, 1):
        for ao, dy in _T1MAP[r]:
            for s in (0, 1):
                for bo, dx in _T1MAP[s]:
                    w6 = w6.at[ao + 1, bo + 1, r, s].set(tw2[:, :, dy, dx])
    w6 = w6.transpose(0, 1, 4, 2, 3, 5).reshape(9, 64, 12)
    b12 = jnp.tile(tb2, 4)
    xt = _call(_k_convt2, B, (u, w6, b12), (True, False, False),
               jax.ShapeDtypeStruct((B, _Mb, 12), f32), True)

    xt = xt.reshape(B, 128, _Wb, 2, 2, 3)[:, :, 1:129]
    x_tilda = xt.transpose(0, 5, 1, 3, 2, 4).reshape(B, 3, 256, 256)
    return (loss, x_tilda, perplexity)


# speedup vs baseline: 1.9070x; 1.9070x over previous
import functools

import jax
import jax.numpy as jnp
from jax import lax
from jax.experimental import pallas as pl
from jax.experimental.pallas import tpu as pltpu

_BETA = 6.6

_P0 = 72
_W = 66
_M = 64 * _W
_L = 2 * _P0 + _M

_T2GRP = {
    -1: ((0, 0, 3),),
    0: ((0, 0, 1), (0, 1, 2), (1, 0, 3)),
    1: ((0, 1, 0), (1, 0, 1), (1, 1, 2)),
    2: ((1, 1, 0),),
}

_S2MAP = {0: (1, -1), 1: (0, 0), 2: (1, 0), 3: (0, 1)}
_T1MAP = {0: ((0, 1), (-1, 3)), 1: ((1, 0), (0, 2))}

_OFF9 = [(ao, bo) for ao in (-1, 0, 1) for bo in (-1, 0, 1)]


def _cparams():
    return pltpu.CompilerParams(
        dimension_semantics=("parallel",),
        vmem_limit_bytes=60 * 1024 * 1024,
    )


def _dot(a, b):
    return jnp.dot(a, b, preferred_element_type=jnp.float32)


def _valid_mask():
    return (lax.broadcasted_iota(jnp.int32, (_M, 1), 0) % _W) < 64


def _store_padded(ref, val, c):
    ref[0:_P0, :] = jnp.zeros((_P0, c), jnp.float32)
    ref[_P0:_P0 + _M, :] = val
    ref[_P0 + _M:_L, :] = jnp.zeros((_L - _P0 - _M, c), jnp.float32)


def _conv3x3_acc(src, wtaps_ref, relu_src):
    acc = None
    for t, (ao, bo) in enumerate(_OFF9):
        st = _P0 + _W * ao + bo
        x = src[st:st + _M, :]
        if relu_src:
            x = jnp.maximum(x, 0.0)
        c = _dot(x, wtaps_ref[t])
        acc = c if acc is None else acc + c
    return acc


def _k_conv1(pat_ref, w_ref, b_ref, o_ref):
    valid = _valid_mask()
    for p in range(4):
        y = jnp.maximum(_dot(pat_ref[0, p], w_ref[...]) + b_ref[...], 0.0)
        _store_padded(o_ref.at[0, p], jnp.where(valid, y, 0.0), 64)


def _k_conv2(in_ref, w_ref, b_ref, o_ref):
    acc = None
    for dy in range(4):
        r, ao = _S2MAP[dy]
        for dx in range(4):
            s, bo = _S2MAP[dx]
            p = r * 2 + s
            st = _P0 + _W * ao + bo
            c = _dot(in_ref[0, p, st:st + _M, :], w_ref[dy * 4 + dx])
            acc = c if acc is None else acc + c
    z = jnp.maximum(acc + b_ref[...], 0.0)
    z = jnp.where(_valid_mask(), z, 0.0)
    _store_padded(o_ref.at[0], z, 128)


def _k_enc_trunk(in_ref, w3_ref, b3_ref, r0a_ref, r0b_ref, r1a_ref, r1b_ref,
                 wp_ref, bp_ref, o_ref, sz_ref, sa_ref):
    valid = _valid_mask()
    z3 = _conv3x3_acc(in_ref.at[0], w3_ref, False) + b3_ref[...]
    _store_padded(sz_ref, jnp.where(valid, z3, 0.0), 128)
    for ra, rb in ((r0a_ref, r0b_ref), (r1a_ref, r1b_ref)):
        ta = _conv3x3_acc(sz_ref, ra, True)
        _store_padded(sa_ref, jnp.where(valid, jnp.maximum(ta, 0.0), 0.0), 32)
        tb = _conv3x3_acc(sa_ref, rb, False)
        sz_ref[_P0:_P0 + _M, :] = (sz_ref[_P0:_P0 + _M, :]
                                   + jnp.where(valid, tb, 0.0))
    h = jnp.maximum(sz_ref[_P0:_P0 + _M, :], 0.0)
    o_ref[0] = _dot(h, wp_ref[...]) + bp_ref[...]


def _k_vq(tok_ref, et_ref, e_ref, q_ref, cnt_ref, se_ref):
    valid = _valid_mask()
    z = tok_ref[0]
    zn = jnp.sum(z * z, axis=1, keepdims=True)
    e2 = jnp.sum(et_ref[...] * et_ref[...], axis=0, keepdims=True)
    d = (zn + e2) - 2.0 * _dot(z, et_ref[...])
    idx = jnp.argmin(d, axis=1)
    oh = jnp.where(
        lax.broadcasted_iota(jnp.int32, (_M, 512), 1) == idx[:, None],
        1.0, 0.0)
    q = _dot(oh, e_ref[...])
    vf = jnp.where(valid, 1.0, 0.0)
    qm = q * vf
    q_ref[0] = qm
    cnt_ref[0] = jnp.broadcast_to(jnp.sum(oh * vf, axis=0, keepdims=True),
                                  (8, 512))
    se = jnp.sum((qm - z * vf) ** 2)
    se_ref[0] = jnp.full((8, 128), se, jnp.float32)


def _k_dec_trunk(q_ref, w1_ref, b1_ref, r0a_ref, r0b_ref, r1a_ref, r1b_ref,
                 o_ref, si_ref, sz_ref, sa_ref):
    valid = _valid_mask()
    _store_padded(si_ref, q_ref[0], 64)
    d1 = _conv3x3_acc(si_ref, w1_ref, False) + b1_ref[...]
    _store_padded(sz_ref, jnp.where(valid, d1, 0.0), 128)
    for ra, rb in ((r0a_ref, r0b_ref), (r1a_ref, r1b_ref)):
        ta = _conv3x3_acc(sz_ref, ra, True)
        _store_padded(sa_ref, jnp.where(valid, jnp.maximum(ta, 0.0), 0.0), 32)
        tb = _conv3x3_acc(sa_ref, rb, False)
        sz_ref[_P0:_P0 + _M, :] = (sz_ref[_P0:_P0 + _M, :]
                                   + jnp.where(valid, tb, 0.0))
    h = jnp.maximum(sz_ref[_P0:_P0 + _M, :], 0.0)
    _store_padded(o_ref.at[0], h, 128)


def _k_convt(in_ref, w1_ref, b1_ref, w7_ref, b7_ref, o_ref, ph_ref):
    valid = _valid_mask()
    for r in (0, 1):
        for s in (0, 1):
            acc = None
            for ao, dy in _T1MAP[r]:
                for bo, dx in _T1MAP[s]:
                    st = _P0 + _W * ao + bo
                    c = _dot(in_ref[0, st:st + _M, :], w1_ref[dy * 4 + dx])
                    acc = c if acc is None else acc + c
            y = jnp.maximum(acc + b1_ref[...], 0.0)
            _store_padded(ph_ref.at[r * 2 + s], jnp.where(valid, y, 0.0), 64)
    acc = None
    for wr in (-1, 0, 1, 2):
        r, ror = wr & 1, wr >> 1
        for wc in (-1, 0, 1, 2):
            s, roc = wc & 1, wc >> 1
            st = _P0 + _W * ror + roc
            c = _dot(ph_ref[r * 2 + s, st:st + _M, :],
                     w7_ref[(wr + 1) * 4 + (wc + 1)])
            acc = c if acc is None else acc + c
    o_ref[0] = acc + b7_ref[...]


def _bspec(shape, batched):
    if batched:
        return pl.BlockSpec((1,) + shape[1:],
                            lambda i: (i,) + (0,) * (len(shape) - 1))
    return pl.BlockSpec(shape, lambda i: (0,) * len(shape))


def _call(body, batch, ins, batched_flags, out_shapes, batched_out, scratch=()):
    in_specs = [_bspec(a.shape, f) for a, f in zip(ins, batched_flags)]
    out_specs = jax.tree.map(
        lambda s, f: _bspec(s.shape, f), out_shapes, batched_out)
    return pl.pallas_call(
        body,
        grid=(batch,),
        in_specs=in_specs,
        out_specs=out_specs,
        out_shape=out_shapes,
        scratch_shapes=list(scratch),
        compiler_params=_cparams(),
    )(*ins)


def kernel(x, w1, b1, w2, b2, w3, b3, r0a, r0b, r1a, r1b, wp, bp, E,
           dw1, db1, dr0a, dr0b, dr1a, dr1b, tw1, tb1, tw2, tb2):
    B = x.shape[0]
    f32 = jnp.float32

    xr = x.reshape(B, 3, 64, 4, 64, 4).transpose(0, 3, 5, 2, 4, 1)
    xr = jnp.pad(xr, ((0, 0), (0, 0), (0, 0), (1, 3), (1, 3), (0, 0)))
    pats = []
    for r in (0, 1):
        for s in (0, 1):
            tl = []
            for dy in range(4):
                vr = 2 * r + dy - 1
                qr, aor = vr % 4, vr // 4
                for dx in range(4):
                    vc = 2 * s + dx - 1
                    qc, boc = vc % 4, vc // 4
                    t = xr[:, qr, qc, 1 + aor:65 + aor, 1 + boc:67 + boc, :]
                    tl.append(t.reshape(B, _M, 3))
            pats.append(jnp.concatenate(tl, axis=-1))
    pat = jnp.stack(pats, axis=1)
    w1m = w1.transpose(2, 3, 1, 0).reshape(48, 64)
    y1 = _call(_k_conv1, B, (pat, w1m, b1), (True, False, False),
               jax.ShapeDtypeStruct((B, 4, _L, 64), f32), True)

    w2m = w2.transpose(2, 3, 1, 0).reshape(16, 64, 128)
    z2 = _call(_k_conv2, B, (y1, w2m, b2), (True, False, False),
               jax.ShapeDtypeStruct((B, _L, 128), f32), True)

    w3m = w3.transpose(2, 3, 1, 0).reshape(9, 128, 128)
    r0am = r0a.transpose(2, 3, 1, 0).reshape(9, 128, 32)
    r0bm = r0b.transpose(2, 3, 1, 0).reshape(9, 32, 128)
    r1am = r1a.transpose(2, 3, 1, 0).reshape(9, 128, 32)
    r1bm = r1b.transpose(2, 3, 1, 0).reshape(9, 32, 128)
    wpm = wp[:, :, 0, 0].T
    tok = _call(_k_enc_trunk, B,
                (z2, w3m, b3, r0am, r0bm, r1am, r1bm, wpm, bp),
                (True,) + (False,) * 8,
                jax.ShapeDtypeStruct((B, _M, 64), f32), True,
                scratch=(pltpu.VMEM((_L, 128), f32),
                         pltpu.VMEM((_L, 32), f32)))

    Et = E.T
    q, cnt, se = _call(
        _k_vq, B, (tok, Et, E), (True, False, False),
        (jax.ShapeDtypeStruct((B, _M, 64), f32),
         jax.ShapeDtypeStruct((B, 8, 512), f32),
         jax.ShapeDtypeStruct((B, 8, 128), f32)),
        (True, True, True))
    n_tok = B * 4096
    counts = jnp.sum(cnt[:, 0, :], axis=0)
    probs = counts / n_tok
    perplexity = jnp.exp(-jnp.sum(probs * jnp.log(probs + 1e-10)))
    loss = (1.0 + _BETA) * jnp.sum(se[:, 0, 0]) / (n_tok * 64)

    dw1m = dw1.transpose(2, 3, 1, 0).reshape(9, 64, 128)
    dr0am = dr0a.transpose(2, 3, 1, 0).reshape(9, 128, 32)
    dr0bm = dr0b.transpose(2, 3, 1, 0).reshape(9, 32, 128)
    dr1am = dr1a.transpose(2, 3, 1, 0).reshape(9, 128, 32)
    dr1bm = dr1b.transpose(2, 3, 1, 0).reshape(9, 32, 128)
    dtr = _call(_k_dec_trunk, B,
                (q, dw1m, db1, dr0am, dr0bm, dr1am, dr1bm),
                (True,) + (False,) * 6,
                jax.ShapeDtypeStruct((B, _L, 128), f32), True,
                scratch=(pltpu.VMEM((_L, 64), f32),
                         pltpu.VMEM((_L, 128), f32),
                         pltpu.VMEM((_L, 32), f32)))

    tw1m = tw1.transpose(2, 3, 0, 1).reshape(16, 128, 64)
    w7 = jnp.zeros((4, 4, 64, 4, 4, 3), f32)
    for wr in (-1, 0, 1, 2):
        for p_r, u_r, dy in _T2GRP[wr]:
            for wc in (-1, 0, 1, 2):
                for p_c, u_c, dx in _T2GRP[wc]:
                    w7 = w7.at[wr + 1, wc + 1, :, 2 * p_r + u_r,
                               2 * p_c + u_c].set(tw2[:, :, dy, dx])
    w7 = w7.reshape(16, 64, 48)
    b48 = jnp.tile(tb2, 16)
    xt = _call(_k_convt, B, (dtr, tw1m, tb1, w7, b48),
               (True,) + (False,) * 4,
               jax.ShapeDtypeStruct((B, _M, 48), f32), True,
               scratch=(pltpu.VMEM((4, _L, 64), f32),))

    xt = xt.reshape(B, 64, _W, 4, 4, 3)[:, :, :64]
    x_tilda = xt.transpose(0, 5, 1, 3, 2, 4).reshape(B, 3, 256, 256)
    return (loss, x_tilda, perplexity)


# speedup vs baseline: 4.5534x; 2.3878x over previous
import functools

import jax
import jax.numpy as jnp
from jax import lax
from jax.experimental import pallas as pl
from jax.experimental.pallas import tpu as pltpu

_BETA = 6.6

_P0 = 72
_W = 66
_M = 64 * _W
_L = 2 * _P0 + _M

_T2GRP = {
    -1: ((0, 0, 3),),
    0: ((0, 0, 1), (0, 1, 2), (1, 0, 3)),
    1: ((0, 1, 0), (1, 0, 1), (1, 1, 2)),
    2: ((1, 1, 0),),
}

_S2MAP = {0: (1, -1), 1: (0, 0), 2: (1, 0), 3: (0, 1)}
_T1MAP = {0: ((0, 1), (-1, 3)), 1: ((1, 0), (0, 2))}

_OFF9 = [(ao, bo) for ao in (-1, 0, 1) for bo in (-1, 0, 1)]


def _cparams():
    return pltpu.CompilerParams(
        dimension_semantics=("parallel",),
        vmem_limit_bytes=60 * 1024 * 1024,
    )


def _dot(a, b):
    return jnp.dot(a, b, preferred_element_type=jnp.float32)


def _valid_mask():
    return (lax.broadcasted_iota(jnp.int32, (_M, 1), 0) % _W) < 64


def _store_padded(ref, val, c):
    ref[0:_P0, :] = jnp.zeros((_P0, c), jnp.float32)
    ref[_P0:_P0 + _M, :] = val
    ref[_P0 + _M:_L, :] = jnp.zeros((_L - _P0 - _M, c), jnp.float32)


def _conv3x3_acc(src, wtaps_ref, relu_src):
    acc = None
    for t, (ao, bo) in enumerate(_OFF9):
        st = _P0 + _W * ao + bo
        x = src[st:st + _M, :]
        if relu_src:
            x = jnp.maximum(x, 0.0)
        c = _dot(x, wtaps_ref[t])
        acc = c if acc is None else acc + c
    return acc


def _k_conv1(xq_ref, w_ref, b_ref, o_ref):
    valid = _valid_mask()
    for r in (0, 1):
        for s in (0, 1):
            rows = []
            for dy in range(4):
                vr = 2 * r + dy - 1
                qr, aor = vr % 4, vr // 4
                for dx in range(4):
                    vc = 2 * s + dx - 1
                    qc, boc = vc % 4, vc // 4
                    q = qr * 4 + qc
                    st = (1 + aor) * _W + (1 + boc)
                    rows.append(xq_ref[0, q * 3:(q + 1) * 3, st:st + _M])
            lhsT = jnp.concatenate(rows, axis=0)
            y = lax.dot_general(lhsT, w_ref[...], (((0,), (0,)), ((), ())),
                                preferred_element_type=jnp.float32)
            y = jnp.maximum(y + b_ref[...], 0.0)
            _store_padded(o_ref.at[0, r * 2 + s], jnp.where(valid, y, 0.0), 64)


def _k_conv2(in_ref, w_ref, b_ref, o_ref):
    acc = None
    for dy in range(4):
        r, ao = _S2MAP[dy]
        for dx in range(4):
            s, bo = _S2MAP[dx]
            p = r * 2 + s
            st = _P0 + _W * ao + bo
            c = _dot(in_ref[0, p, st:st + _M, :], w_ref[dy * 4 + dx])
            acc = c if acc is None else acc + c
    z = jnp.maximum(acc + b_ref[...], 0.0)
    z = jnp.where(_valid_mask(), z, 0.0)
    _store_padded(o_ref.at[0], z, 128)


def _k_enc_trunk(in_ref, w3_ref, b3_ref, r0a_ref, r0b_ref, r1a_ref, r1b_ref,
                 wp_ref, bp_ref, o_ref, sz_ref, sa_ref):
    valid = _valid_mask()
    z3 = _conv3x3_acc(in_ref.at[0], w3_ref, False) + b3_ref[...]
    _store_padded(sz_ref, jnp.where(valid, z3, 0.0), 128)
    for ra, rb in ((r0a_ref, r0b_ref), (r1a_ref, r1b_ref)):
        ta = _conv3x3_acc(sz_ref, ra, True)
        _store_padded(sa_ref, jnp.where(valid, jnp.maximum(ta, 0.0), 0.0), 32)
        tb = _conv3x3_acc(sa_ref, rb, False)
        sz_ref[_P0:_P0 + _M, :] = (sz_ref[_P0:_P0 + _M, :]
                                   + jnp.where(valid, tb, 0.0))
    h = jnp.maximum(sz_ref[_P0:_P0 + _M, :], 0.0)
    o_ref[0] = _dot(h, wp_ref[...]) + bp_ref[...]


def _k_vq(tok_ref, et_ref, e_ref, q_ref, cnt_ref, se_ref):
    valid = _valid_mask()
    z = tok_ref[0]
    zn = jnp.sum(z * z, axis=1, keepdims=True)
    e2 = jnp.sum(et_ref[...] * et_ref[...], axis=0, keepdims=True)
    d = (zn + e2) - 2.0 * _dot(z, et_ref[...])
    idx = jnp.argmin(d, axis=1)
    oh = jnp.where(
        lax.broadcasted_iota(jnp.int32, (_M, 512), 1) == idx[:, None],
        1.0, 0.0)
    q = _dot(oh, e_ref[...])
    vf = jnp.where(valid, 1.0, 0.0)
    qm = q * vf
    q_ref[0] = qm
    cnt_ref[0] = jnp.broadcast_to(jnp.sum(oh * vf, axis=0, keepdims=True),
                                  (8, 512))
    se = jnp.sum((qm - z * vf) ** 2)
    se_ref[0] = jnp.full((8, 128), se, jnp.float32)


def _k_dec_trunk(q_ref, w1_ref, b1_ref, r0a_ref, r0b_ref, r1a_ref, r1b_ref,
                 o_ref, si_ref, sz_ref, sa_ref):
    valid = _valid_mask()
    _store_padded(si_ref, q_ref[0], 64)
    d1 = _conv3x3_acc(si_ref, w1_ref, False) + b1_ref[...]
    _store_padded(sz_ref, jnp.where(valid, d1, 0.0), 128)
    for ra, rb in ((r0a_ref, r0b_ref), (r1a_ref, r1b_ref)):
        ta = _conv3x3_acc(sz_ref, ra, True)
        _store_padded(sa_ref, jnp.where(valid, jnp.maximum(ta, 0.0), 0.0), 32)
        tb = _conv3x3_acc(sa_ref, rb, False)
        sz_ref[_P0:_P0 + _M, :] = (sz_ref[_P0:_P0 + _M, :]
                                   + jnp.where(valid, tb, 0.0))
    h = jnp.maximum(sz_ref[_P0:_P0 + _M, :], 0.0)
    _store_padded(o_ref.at[0], h, 128)


def _k_convt(in_ref, w1_ref, b1_ref, w7_ref, b7_ref, o_ref, ph_ref):
    valid = _valid_mask()
    for r in (0, 1):
        for s in (0, 1):
            acc = None
            for ao, dy in _T1MAP[r]:
                for bo, dx in _T1MAP[s]:
                    st = _P0 + _W * ao + bo
                    c = _dot(in_ref[0, st:st + _M, :], w1_ref[dy * 4 + dx])
                    acc = c if acc is None else acc + c
            y = jnp.maximum(acc + b1_ref[...], 0.0)
            _store_padded(ph_ref.at[r * 2 + s], jnp.where(valid, y, 0.0), 64)
    acc = None
    for wr in (-1, 0, 1, 2):
        r, ror = wr & 1, wr >> 1
        for wc in (-1, 0, 1, 2):
            s, roc = wc & 1, wc >> 1
            st = _P0 + _W * ror + roc
            c = _dot(ph_ref[r * 2 + s, st:st + _M, :],
                     w7_ref[(wr + 1) * 4 + (wc + 1)])
            acc = c if acc is None else acc + c
    o_ref[0] = acc + b7_ref[...]


def _bspec(shape, batched):
    if batched:
        return pl.BlockSpec((1,) + shape[1:],
                            lambda i: (i,) + (0,) * (len(shape) - 1))
    return pl.BlockSpec(shape, lambda i: (0,) * len(shape))


def _call(body, batch, ins, batched_flags, out_shapes, batched_out, scratch=()):
    in_specs = [_bspec(a.shape, f) for a, f in zip(ins, batched_flags)]
    out_specs = jax.tree.map(
        lambda s, f: _bspec(s.shape, f), out_shapes, batched_out)
    return pl.pallas_call(
        body,
        grid=(batch,),
        in_specs=in_specs,
        out_specs=out_specs,
        out_shape=out_shapes,
        scratch_shapes=list(scratch),
        compiler_params=_cparams(),
    )(*ins)


def kernel(x, w1, b1, w2, b2, w3, b3, r0a, r0b, r1a, r1b, wp, bp, E,
           dw1, db1, dr0a, dr0b, dr1a, dr1b, tw1, tb1, tw2, tb2):
    B = x.shape[0]
    f32 = jnp.float32

    xq = x.reshape(B, 3, 64, 4, 64, 4).transpose(0, 1, 3, 5, 2, 4)
    xq = jnp.pad(xq, ((0, 0), (0, 0), (0, 0), (0, 0), (1, 3), (1, 1)))
    xq = xq.transpose(0, 2, 3, 1, 4, 5).reshape(B, 48, 68 * _W)
    w1m = w1.transpose(2, 3, 1, 0).reshape(48, 64)
    y1 = _call(_k_conv1, B, (xq, w1m, b1), (True, False, False),
               jax.ShapeDtypeStruct((B, 4, _L, 64), f32), True)

    w2m = w2.transpose(2, 3, 1, 0).reshape(16, 64, 128)
    z2 = _call(_k_conv2, B, (y1, w2m, b2), (True, False, False),
               jax.ShapeDtypeStruct((B, _L, 128), f32), True)

    w3m = w3.transpose(2, 3, 1, 0).reshape(9, 128, 128)
    r0am = r0a.transpose(2, 3, 1, 0).reshape(9, 128, 32)
    r0bm = r0b.transpose(2, 3, 1, 0).reshape(9, 32, 128)
    r1am = r1a.transpose(2, 3, 1, 0).reshape(9, 128, 32)
    r1bm = r1b.transpose(2, 3, 1, 0).reshape(9, 32, 128)
    wpm = wp[:, :, 0, 0].T
    tok = _call(_k_enc_trunk, B,
                (z2, w3m, b3, r0am, r0bm, r1am, r1bm, wpm, bp),
                (True,) + (False,) * 8,
                jax.ShapeDtypeStruct((B, _M, 64), f32), True,
                scratch=(pltpu.VMEM((_L, 128), f32),
                         pltpu.VMEM((_L, 32), f32)))

    Et = E.T
    q, cnt, se = _call(
        _k_vq, B, (tok, Et, E), (True, False, False),
        (jax.ShapeDtypeStruct((B, _M, 64), f32),
         jax.ShapeDtypeStruct((B, 8, 512), f32),
         jax.ShapeDtypeStruct((B, 8, 128), f32)),
        (True, True, True))
    n_tok = B * 4096
    counts = jnp.sum(cnt[:, 0, :], axis=0)
    probs = counts / n_tok
    perplexity = jnp.exp(-jnp.sum(probs * jnp.log(probs + 1e-10)))
    loss = (1.0 + _BETA) * jnp.sum(se[:, 0, 0]) / (n_tok * 64)

    dw1m = dw1.transpose(2, 3, 1, 0).reshape(9, 64, 128)
    dr0am = dr0a.transpose(2, 3, 1, 0).reshape(9, 128, 32)
    dr0bm = dr0b.transpose(2, 3, 1, 0).reshape(9, 32, 128)
    dr1am = dr1a.transpose(2, 3, 1, 0).reshape(9, 128, 32)
    dr1bm = dr1b.transpose(2, 3, 1, 0).reshape(9, 32, 128)
    dtr = _call(_k_dec_trunk, B,
                (q, dw1m, db1, dr0am, dr0bm, dr1am, dr1bm),
                (True,) + (False,) * 6,
                jax.ShapeDtypeStruct((B, _L, 128), f32), True,
                scratch=(pltpu.VMEM((_L, 64), f32),
                         pltpu.VMEM((_L, 128), f32),
                         pltpu.VMEM((_L, 32), f32)))

    tw1m = tw1.transpose(2, 3, 0, 1).reshape(16, 128, 64)
    w7 = jnp.zeros((4, 4, 64, 4, 4, 3), f32)
    for wr in (-1, 0, 1, 2):
        for p_r, u_r, dy in _T2GRP[wr]:
            for wc in (-1, 0, 1, 2):
                for p_c, u_c, dx in _T2GRP[wc]:
                    w7 = w7.at[wr + 1, wc + 1, :, 2 * p_r + u_r,
                               2 * p_c + u_c].set(tw2[:, :, dy, dx])
    w7 = w7.reshape(16, 64, 48)
    b48 = jnp.tile(tb2, 16)
    xt = _call(_k_convt, B, (dtr, tw1m, tb1, w7, b48),
               (True,) + (False,) * 4,
               jax.ShapeDtypeStruct((B, _M, 48), f32), True,
               scratch=(pltpu.VMEM((4, _L, 64), f32),))

    xt = xt.reshape(B, 64, _W, 4, 4, 3)[:, :, :64]
    x_tilda = xt.transpose(0, 5, 1, 3, 2, 4).reshape(B, 3, 256, 256)
    return (loss, x_tilda, perplexity)


# speedup vs baseline: 4.9476x; 1.0866x over previous
import functools

import jax
import jax.numpy as jnp
from jax import lax
from jax.experimental import pallas as pl
from jax.experimental.pallas import tpu as pltpu

_BETA = 6.6

_P0 = 72
_W = 66
_M = 64 * _W
_L = 2 * _P0 + _M

_T2GRP = {
    -1: ((0, 0, 3),),
    0: ((0, 0, 1), (0, 1, 2), (1, 0, 3)),
    1: ((0, 1, 0), (1, 0, 1), (1, 1, 2)),
    2: ((1, 1, 0),),
}

_S2MAP = {0: (1, -1), 1: (0, 0), 2: (1, 0), 3: (0, 1)}
_T1MAP = {0: ((0, 1), (-1, 3)), 1: ((1, 0), (0, 2))}

_OFF9 = [(ao, bo) for ao in (-1, 0, 1) for bo in (-1, 0, 1)]


def _cparams():
    return pltpu.CompilerParams(
        dimension_semantics=("parallel",),
        vmem_limit_bytes=60 * 1024 * 1024,
    )


def _dot(a, b):
    return jnp.dot(a, b, preferred_element_type=jnp.float32)


def _valid_mask():
    return (lax.broadcasted_iota(jnp.int32, (_M, 1), 0) % _W) < 64


def _store_padded(ref, val, c):
    ref[0:_P0, :] = jnp.zeros((_P0, c), jnp.float32)
    ref[_P0:_P0 + _M, :] = val
    ref[_P0 + _M:_L, :] = jnp.zeros((_L - _P0 - _M, c), jnp.float32)


def _conv3x3_acc(src, wtaps_ref, relu_src):
    acc = None
    for t, (ao, bo) in enumerate(_OFF9):
        st = _P0 + _W * ao + bo
        x = src[st:st + _M, :]
        if relu_src:
            x = jnp.maximum(x, 0.0)
        c = _dot(x, wtaps_ref[t])
        acc = c if acc is None else acc + c
    return acc


def _conv3x3_cat(src, wcat_ref, cat_ref, cin, relu_src):
    for t, (ao, bo) in enumerate(_OFF9):
        st = _P0 + _W * ao + bo
        x = src[st:st + _M, :]
        if relu_src:
            x = jnp.maximum(x, 0.0)
        cat_ref[:, t * cin:(t + 1) * cin] = x
    return _dot(cat_ref[:, :9 * cin], wcat_ref[...])


def _k_conv1(xq_ref, w_ref, b_ref, o_ref):
    valid = _valid_mask()
    for r in (0, 1):
        for s in (0, 1):
            rows = []
            for dy in range(4):
                vr = 2 * r + dy - 1
                qr, aor = vr % 4, vr // 4
                for dx in range(4):
                    vc = 2 * s + dx - 1
                    qc, boc = vc % 4, vc // 4
                    q = qr * 4 + qc
                    st = (1 + aor) * _W + (1 + boc)
                    rows.append(xq_ref[0, q * 3:(q + 1) * 3, st:st + _M])
            lhsT = jnp.concatenate(rows, axis=0)
            y = lax.dot_general(lhsT, w_ref[...], (((0,), (0,)), ((), ())),
                                preferred_element_type=jnp.float32)
            y = jnp.maximum(y + b_ref[...], 0.0)
            _store_padded(o_ref.at[0, r * 2 + s], jnp.where(valid, y, 0.0), 64)


def _k_conv2(in_ref, w_ref, b_ref, o_ref):
    acc = None
    for dy in range(4):
        r, ao = _S2MAP[dy]
        for dx in range(4):
            s, bo = _S2MAP[dx]
            p = r * 2 + s
            st = _P0 + _W * ao + bo
            c = _dot(in_ref[0, p, st:st + _M, :], w_ref[dy * 4 + dx])
            acc = c if acc is None else acc + c
    z = jnp.maximum(acc + b_ref[...], 0.0)
    z = jnp.where(_valid_mask(), z, 0.0)
    _store_padded(o_ref.at[0], z, 128)


def _k_enc_trunk(in_ref, w3_ref, b3_ref, r0a_ref, r0b_ref, r1a_ref, r1b_ref,
                 wp_ref, bp_ref, o_ref, sz_ref, sa_ref, cat_ref):
    valid = _valid_mask()
    z3 = _conv3x3_cat(in_ref.at[0], w3_ref, cat_ref, 128, False) + b3_ref[...]
    _store_padded(sz_ref, jnp.where(valid, z3, 0.0), 128)
    for ra, rb in ((r0a_ref, r0b_ref), (r1a_ref, r1b_ref)):
        ta = _conv3x3_cat(sz_ref, ra, cat_ref, 128, True)
        _store_padded(sa_ref, jnp.where(valid, jnp.maximum(ta, 0.0), 0.0), 32)
        tb = _conv3x3_cat(sa_ref, rb, cat_ref, 32, False)
        sz_ref[_P0:_P0 + _M, :] = (sz_ref[_P0:_P0 + _M, :]
                                   + jnp.where(valid, tb, 0.0))
    h = jnp.maximum(sz_ref[_P0:_P0 + _M, :], 0.0)
    o_ref[0] = _dot(h, wp_ref[...]) + bp_ref[...]


def _k_vq(tok_ref, et_ref, e_ref, q_ref, cnt_ref, se_ref):
    valid = _valid_mask()
    z = tok_ref[0]
    zn = jnp.sum(z * z, axis=1, keepdims=True)
    e2 = jnp.sum(et_ref[...] * et_ref[...], axis=0, keepdims=True)
    d = (zn + e2) - 2.0 * _dot(z, et_ref[...])
    idx = jnp.argmin(d, axis=1)
    oh = jnp.where(
        lax.broadcasted_iota(jnp.int32, (_M, 512), 1) == idx[:, None],
        1.0, 0.0)
    q = _dot(oh, e_ref[...])
    vf = jnp.where(valid, 1.0, 0.0)
    qm = q * vf
    q_ref[0] = qm
    cnt_ref[0] = jnp.broadcast_to(jnp.sum(oh * vf, axis=0, keepdims=True),
                                  (8, 512))
    se = jnp.sum((qm - z * vf) ** 2)
    se_ref[0] = jnp.full((8, 128), se, jnp.float32)


def _k_dec_trunk(q_ref, w1_ref, b1_ref, r0a_ref, r0b_ref, r1a_ref, r1b_ref,
                 o_ref, si_ref, sz_ref, sa_ref, cat_ref):
    valid = _valid_mask()
    _store_padded(si_ref, q_ref[0], 64)
    d1 = _conv3x3_cat(si_ref, w1_ref, cat_ref, 64, False) + b1_ref[...]
    _store_padded(sz_ref, jnp.where(valid, d1, 0.0), 128)
    for ra, rb in ((r0a_ref, r0b_ref), (r1a_ref, r1b_ref)):
        ta = _conv3x3_cat(sz_ref, ra, cat_ref, 128, True)
        _store_padded(sa_ref, jnp.where(valid, jnp.maximum(ta, 0.0), 0.0), 32)
        tb = _conv3x3_cat(sa_ref, rb, cat_ref, 32, False)
        sz_ref[_P0:_P0 + _M, :] = (sz_ref[_P0:_P0 + _M, :]
                                   + jnp.where(valid, tb, 0.0))
    h = jnp.maximum(sz_ref[_P0:_P0 + _M, :], 0.0)
    _store_padded(o_ref.at[0], h, 128)


def _k_convt(in_ref, w1_ref, b1_ref, w7_ref, b7_ref, o_ref, ph_ref):
    valid = _valid_mask()
    for r in (0, 1):
        for s in (0, 1):
            acc = None
            for ao, dy in _T1MAP[r]:
                for bo, dx in _T1MAP[s]:
                    st = _P0 + _W * ao + bo
                    c = _dot(in_ref[0, st:st + _M, :], w1_ref[dy * 4 + dx])
                    acc = c if acc is None else acc + c
            y = jnp.maximum(acc + b1_ref[...], 0.0)
            _store_padded(ph_ref.at[r * 2 + s], jnp.where(valid, y, 0.0), 64)
    acc = None
    for wr in (-1, 0, 1, 2):
        r, ror = wr & 1, wr >> 1
        for wc in (-1, 0, 1, 2):
            s, roc = wc & 1, wc >> 1
            st = _P0 + _W * ror + roc
            c = _dot(ph_ref[r * 2 + s, st:st + _M, :],
                     w7_ref[(wr + 1) * 4 + (wc + 1)])
            acc = c if acc is None else acc + c
    o_ref[0] = acc + b7_ref[...]


def _bspec(shape, batched):
    if batched:
        return pl.BlockSpec((1,) + shape[1:],
                            lambda i: (i,) + (0,) * (len(shape) - 1))
    return pl.BlockSpec(shape, lambda i: (0,) * len(shape))


def _call(body, batch, ins, batched_flags, out_shapes, batched_out, scratch=()):
    in_specs = [_bspec(a.shape, f) for a, f in zip(ins, batched_flags)]
    out_specs = jax.tree.map(
        lambda s, f: _bspec(s.shape, f), out_shapes, batched_out)
    return pl.pallas_call(
        body,
        grid=(batch,),
        in_specs=in_specs,
        out_specs=out_specs,
        out_shape=out_shapes,
        scratch_shapes=list(scratch),
        compiler_params=_cparams(),
    )(*ins)


def kernel(x, w1, b1, w2, b2, w3, b3, r0a, r0b, r1a, r1b, wp, bp, E,
           dw1, db1, dr0a, dr0b, dr1a, dr1b, tw1, tb1, tw2, tb2):
    B = x.shape[0]
    f32 = jnp.float32

    xq = x.reshape(B, 3, 64, 4, 64, 4).transpose(0, 1, 3, 5, 2, 4)
    xq = jnp.pad(xq, ((0, 0), (0, 0), (0, 0), (0, 0), (1, 3), (1, 1)))
    xq = xq.transpose(0, 2, 3, 1, 4, 5).reshape(B, 48, 68 * _W)
    w1m = w1.transpose(2, 3, 1, 0).reshape(48, 64)
    y1 = _call(_k_conv1, B, (xq, w1m, b1), (True, False, False),
               jax.ShapeDtypeStruct((B, 4, _L, 64), f32), True)

    w2m = w2.transpose(2, 3, 1, 0).reshape(16, 64, 128)
    z2 = _call(_k_conv2, B, (y1, w2m, b2), (True, False, False),
               jax.ShapeDtypeStruct((B, _L, 128), f32), True)

    w3m = w3.transpose(2, 3, 1, 0).reshape(9 * 128, 128)
    r0am = r0a.transpose(2, 3, 1, 0).reshape(9 * 128, 32)
    r0bm = r0b.transpose(2, 3, 1, 0).reshape(9 * 32, 128)
    r1am = r1a.transpose(2, 3, 1, 0).reshape(9 * 128, 32)
    r1bm = r1b.transpose(2, 3, 1, 0).reshape(9 * 32, 128)
    wpm = wp[:, :, 0, 0].T
    tok = _call(_k_enc_trunk, B,
                (z2, w3m, b3, r0am, r0bm, r1am, r1bm, wpm, bp),
                (True,) + (False,) * 8,
                jax.ShapeDtypeStruct((B, _M, 64), f32), True,
                scratch=(pltpu.VMEM((_L, 128), f32),
                         pltpu.VMEM((_L, 32), f32),
                         pltpu.VMEM((_M, 9 * 128), f32)))

    Et = E.T
    q, cnt, se = _call(
        _k_vq, B, (tok, Et, E), (True, False, False),
        (jax.ShapeDtypeStruct((B, _M, 64), f32),
         jax.ShapeDtypeStruct((B, 8, 512), f32),
         jax.ShapeDtypeStruct((B, 8, 128), f32)),
        (True, True, True))
    n_tok = B * 4096
    counts = jnp.sum(cnt[:, 0, :], axis=0)
    probs = counts / n_tok
    perplexity = jnp.exp(-jnp.sum(probs * jnp.log(probs + 1e-10)))
    loss = (1.0 + _BETA) * jnp.sum(se[:, 0, 0]) / (n_tok * 64)

    dw1m = dw1.transpose(2, 3, 1, 0).reshape(9 * 64, 128)
    dr0am = dr0a.transpose(2, 3, 1, 0).reshape(9 * 128, 32)
    dr0bm = dr0b.transpose(2, 3, 1, 0).reshape(9 * 32, 128)
    dr1am = dr1a.transpose(2, 3, 1, 0).reshape(9 * 128, 32)
    dr1bm = dr1b.transpose(2, 3, 1, 0).reshape(9 * 32, 128)
    dtr = _call(_k_dec_trunk, B,
                (q, dw1m, db1, dr0am, dr0bm, dr1am, dr1bm),
                (True,) + (False,) * 6,
                jax.ShapeDtypeStruct((B, _L, 128), f32), True,
                scratch=(pltpu.VMEM((_L, 64), f32),
                         pltpu.VMEM((_L, 128), f32),
                         pltpu.VMEM((_L, 32), f32),
                         pltpu.VMEM((_M, 9 * 128), f32)))

    tw1m = tw1.transpose(2, 3, 0, 1).reshape(16, 128, 64)
    w7 = jnp.zeros((4, 4, 64, 4, 4, 3), f32)
    for wr in (-1, 0, 1, 2):
        for p_r, u_r, dy in _T2GRP[wr]:
            for wc in (-1, 0, 1, 2):
                for p_c, u_c, dx in _T2GRP[wc]:
                    w7 = w7.at[wr + 1, wc + 1, :, 2 * p_r + u_r,
                               2 * p_c + u_c].set(tw2[:, :, dy, dx])
    w7 = w7.reshape(16, 64, 48)
    b48 = jnp.tile(tb2, 16)
    xt = _call(_k_convt, B, (dtr, tw1m, tb1, w7, b48),
               (True,) + (False,) * 4,
               jax.ShapeDtypeStruct((B, _M, 48), f32), True,
               scratch=(pltpu.VMEM((4, _L, 64), f32),))

    xt = xt.reshape(B, 64, _W, 4, 4, 3)[:, :, :64]
    x_tilda = xt.transpose(0, 5, 1, 3, 2, 4).reshape(B, 3, 256, 256)
    return (loss, x_tilda, perplexity)


# speedup vs baseline: 5.0818x; 1.0271x over previous
import functools

import jax
import jax.numpy as jnp
from jax import lax
from jax.experimental import pallas as pl
from jax.experimental.pallas import tpu as pltpu

_BETA = 6.6

_P0 = 72
_W = 66
_M = 64 * _W
_L = 2 * _P0 + _M

_T2GRP = {
    -1: ((0, 0, 3),),
    0: ((0, 0, 1), (0, 1, 2), (1, 0, 3)),
    1: ((0, 1, 0), (1, 0, 1), (1, 1, 2)),
    2: ((1, 1, 0),),
}

_S2MAP = {0: (1, -1), 1: (0, 0), 2: (1, 0), 3: (0, 1)}
_T1MAP = {0: ((0, 1), (-1, 3)), 1: ((1, 0), (0, 2))}

_OFF9 = [(ao, bo) for ao in (-1, 0, 1) for bo in (-1, 0, 1)]


def _cparams():
    return pltpu.CompilerParams(
        dimension_semantics=("parallel",),
        vmem_limit_bytes=60 * 1024 * 1024,
    )


def _dot(a, b):
    return jnp.dot(a, b, preferred_element_type=jnp.float32)


def _valid_mask():
    return (lax.broadcasted_iota(jnp.int32, (_M, 1), 0) % _W) < 64


def _store_padded(ref, val, c):
    ref[0:_P0, :] = jnp.zeros((_P0, c), jnp.float32)
    ref[_P0:_P0 + _M, :] = val
    ref[_P0 + _M:_L, :] = jnp.zeros((_L - _P0 - _M, c), jnp.float32)


def _conv3x3_acc(src, wtaps_ref, relu_src):
    acc = None
    for t, (ao, bo) in enumerate(_OFF9):
        st = _P0 + _W * ao + bo
        x = src[st:st + _M, :]
        if relu_src:
            x = jnp.maximum(x, 0.0)
        c = _dot(x, wtaps_ref[t])
        acc = c if acc is None else acc + c
    return acc


def _conv3x3_cat(src, wcat_ref, cat_ref, cin, relu_src):
    for t, (ao, bo) in enumerate(_OFF9):
        st = _P0 + _W * ao + bo
        x = src[st:st + _M, :]
        if relu_src:
            x = jnp.maximum(x, 0.0)
        cat_ref[:, t * cin:(t + 1) * cin] = x
    return _dot(cat_ref[:, :9 * cin], wcat_ref[...])


def _k_conv1(xq_ref, w_ref, b_ref, o_ref):
    valid = _valid_mask()
    for r in (0, 1):
        for s in (0, 1):
            rows = []
            for dy in range(4):
                vr = 2 * r + dy - 1
                qr, aor = vr % 4, vr // 4
                for dx in range(4):
                    vc = 2 * s + dx - 1
                    qc, boc = vc % 4, vc // 4
                    q = qr * 4 + qc
                    st = (1 + aor) * _W + (1 + boc)
                    rows.append(xq_ref[0, q * 3:(q + 1) * 3, st:st + _M])
            lhsT = jnp.concatenate(rows, axis=0)
            y = lax.dot_general(lhsT, w_ref[...], (((0,), (0,)), ((), ())),
                                preferred_element_type=jnp.float32)
            y = jnp.maximum(y + b_ref[...], 0.0)
            _store_padded(o_ref.at[0, r * 2 + s], jnp.where(valid, y, 0.0), 64)


def _k_conv2(in_ref, w_ref, b_ref, o_ref, cat_ref):
    for dy in range(4):
        r, ao = _S2MAP[dy]
        for dx in range(4):
            s, bo = _S2MAP[dx]
            p = r * 2 + s
            st = _P0 + _W * ao + bo
            t = dy * 4 + dx
            cat_ref[:, t * 64:(t + 1) * 64] = in_ref[0, p, st:st + _M, :]
    z = jnp.maximum(_dot(cat_ref[...], w_ref[...]) + b_ref[...], 0.0)
    z = jnp.where(_valid_mask(), z, 0.0)
    _store_padded(o_ref.at[0], z, 128)


def _k_enc_trunk(in_ref, w3_ref, b3_ref, r0a_ref, r0b_ref, r1a_ref, r1b_ref,
                 wp_ref, bp_ref, o_ref, sz_ref, sa_ref, cat_ref):
    valid = _valid_mask()
    z3 = _conv3x3_cat(in_ref.at[0], w3_ref, cat_ref, 128, False) + b3_ref[...]
    _store_padded(sz_ref, jnp.where(valid, z3, 0.0), 128)
    for ra, rb in ((r0a_ref, r0b_ref), (r1a_ref, r1b_ref)):
        ta = _conv3x3_cat(sz_ref, ra, cat_ref, 128, True)
        _store_padded(sa_ref, jnp.where(valid, jnp.maximum(ta, 0.0), 0.0), 32)
        tb = _conv3x3_cat(sa_ref, rb, cat_ref, 32, False)
        sz_ref[_P0:_P0 + _M, :] = (sz_ref[_P0:_P0 + _M, :]
                                   + jnp.where(valid, tb, 0.0))
    h = jnp.maximum(sz_ref[_P0:_P0 + _M, :], 0.0)
    o_ref[0] = _dot(h, wp_ref[...]) + bp_ref[...]


def _k_vq(tok_ref, et_ref, e_ref, q_ref, cnt_ref, se_ref):
    valid = _valid_mask()
    z = tok_ref[0]
    zn = jnp.sum(z * z, axis=1, keepdims=True)
    e2 = jnp.sum(et_ref[...] * et_ref[...], axis=0, keepdims=True)
    d = (zn + e2) - 2.0 * _dot(z, et_ref[...])
    idx = jnp.argmin(d, axis=1)
    oh = jnp.where(
        lax.broadcasted_iota(jnp.int32, (_M, 512), 1) == idx[:, None],
        1.0, 0.0)
    q = _dot(oh, e_ref[...])
    vf = jnp.where(valid, 1.0, 0.0)
    qm = q * vf
    q_ref[0] = qm
    cnt_ref[0] = jnp.broadcast_to(jnp.sum(oh * vf, axis=0, keepdims=True),
                                  (8, 512))
    se = jnp.sum((qm - z * vf) ** 2)
    se_ref[0] = jnp.full((8, 128), se, jnp.float32)


def _k_dec_trunk(q_ref, w1_ref, b1_ref, r0a_ref, r0b_ref, r1a_ref, r1b_ref,
                 o_ref, si_ref, sz_ref, sa_ref, cat_ref):
    valid = _valid_mask()
    _store_padded(si_ref, q_ref[0], 64)
    d1 = _conv3x3_cat(si_ref, w1_ref, cat_ref, 64, False) + b1_ref[...]
    _store_padded(sz_ref, jnp.where(valid, d1, 0.0), 128)
    for ra, rb in ((r0a_ref, r0b_ref), (r1a_ref, r1b_ref)):
        ta = _conv3x3_cat(sz_ref, ra, cat_ref, 128, True)
        _store_padded(sa_ref, jnp.where(valid, jnp.maximum(ta, 0.0), 0.0), 32)
        tb = _conv3x3_cat(sa_ref, rb, cat_ref, 32, False)
        sz_ref[_P0:_P0 + _M, :] = (sz_ref[_P0:_P0 + _M, :]
                                   + jnp.where(valid, tb, 0.0))
    h = jnp.maximum(sz_ref[_P0:_P0 + _M, :], 0.0)
    _store_padded(o_ref.at[0], h, 128)


def _k_convt(in_ref, w1_ref, b1_ref, w7_ref, b7_ref, o_ref, ph_ref, cat_ref):
    valid = _valid_mask()
    for r in (0, 1):
        for s in (0, 1):
            for t, ((ao, dy), (bo, dx)) in enumerate(
                    (a, b) for a in _T1MAP[r] for b in _T1MAP[s]):
                st = _P0 + _W * ao + bo
                cat_ref[:, t * 128:(t + 1) * 128] = in_ref[0, st:st + _M, :]
            y = _dot(cat_ref[:, :512], w1_ref[r * 2 + s])
            y = jnp.maximum(y + b1_ref[...], 0.0)
            _store_padded(ph_ref.at[r * 2 + s], jnp.where(valid, y, 0.0), 64)
    for i, wr in enumerate((-1, 0, 1, 2)):
        r, ror = wr & 1, wr >> 1
        for j, wc in enumerate((-1, 0, 1, 2)):
            s, roc = wc & 1, wc >> 1
            st = _P0 + _W * ror + roc
            t = i * 4 + j
            cat_ref[:, t * 64:(t + 1) * 64] = ph_ref[r * 2 + s, st:st + _M, :]
    o_ref[0] = _dot(cat_ref[...], w7_ref[...]) + b7_ref[...]


def _bspec(shape, batched):
    if batched:
        return pl.BlockSpec((1,) + shape[1:],
                            lambda i: (i,) + (0,) * (len(shape) - 1))
    return pl.BlockSpec(shape, lambda i: (0,) * len(shape))


def _call(body, batch, ins, batched_flags, out_shapes, batched_out, scratch=()):
    in_specs = [_bspec(a.shape, f) for a, f in zip(ins, batched_flags)]
    out_specs = jax.tree.map(
        lambda s, f: _bspec(s.shape, f), out_shapes, batched_out)
    return pl.pallas_call(
        body,
        grid=(batch,),
        in_specs=in_specs,
        out_specs=out_specs,
        out_shape=out_shapes,
        scratch_shapes=list(scratch),
        compiler_params=_cparams(),
    )(*ins)


def kernel(x, w1, b1, w2, b2, w3, b3, r0a, r0b, r1a, r1b, wp, bp, E,
           dw1, db1, dr0a, dr0b, dr1a, dr1b, tw1, tb1, tw2, tb2):
    B = x.shape[0]
    f32 = jnp.float32

    xq = x.reshape(B, 3, 64, 4, 64, 4).transpose(0, 1, 3, 5, 2, 4)
    xq = jnp.pad(xq, ((0, 0), (0, 0), (0, 0), (0, 0), (1, 3), (1, 1)))
    xq = xq.transpose(0, 2, 3, 1, 4, 5).reshape(B, 48, 68 * _W)
    w1m = w1.transpose(2, 3, 1, 0).reshape(48, 64)
    y1 = _call(_k_conv1, B, (xq, w1m, b1), (True, False, False),
               jax.ShapeDtypeStruct((B, 4, _L, 64), f32), True)

    w2m = w2.transpose(2, 3, 1, 0).reshape(16 * 64, 128)
    z2 = _call(_k_conv2, B, (y1, w2m, b2), (True, False, False),
               jax.ShapeDtypeStruct((B, _L, 128), f32), True,
               scratch=(pltpu.VMEM((_M, 16 * 64), f32),))

    w3m = w3.transpose(2, 3, 1, 0).reshape(9 * 128, 128)
    r0am = r0a.transpose(2, 3, 1, 0).reshape(9 * 128, 32)
    r0bm = r0b.transpose(2, 3, 1, 0).reshape(9 * 32, 128)
    r1am = r1a.transpose(2, 3, 1, 0).reshape(9 * 128, 32)
    r1bm = r1b.transpose(2, 3, 1, 0).reshape(9 * 32, 128)
    wpm = wp[:, :, 0, 0].T
    tok = _call(_k_enc_trunk, B,
                (z2, w3m, b3, r0am, r0bm, r1am, r1bm, wpm, bp),
                (True,) + (False,) * 8,
                jax.ShapeDtypeStruct((B, _M, 64), f32), True,
                scratch=(pltpu.VMEM((_L, 128), f32),
                         pltpu.VMEM((_L, 32), f32),
                         pltpu.VMEM((_M, 9 * 128), f32)))

    Et = E.T
    q, cnt, se = _call(
        _k_vq, B, (tok, Et, E), (True, False, False),
        (jax.ShapeDtypeStruct((B, _M, 64), f32),
         jax.ShapeDtypeStruct((B, 8, 512), f32),
         jax.ShapeDtypeStruct((B, 8, 128), f32)),
        (True, True, True))
    n_tok = B * 4096
    counts = jnp.sum(cnt[:, 0, :], axis=0)
    probs = counts / n_tok
    perplexity = jnp.exp(-jnp.sum(probs * jnp.log(probs + 1e-10)))
    loss = (1.0 + _BETA) * jnp.sum(se[:, 0, 0]) / (n_tok * 64)

    dw1m = dw1.transpose(2, 3, 1, 0).reshape(9 * 64, 128)
    dr0am = dr0a.transpose(2, 3, 1, 0).reshape(9 * 128, 32)
    dr0bm = dr0b.transpose(2, 3, 1, 0).reshape(9 * 32, 128)
    dr1am = dr1a.transpose(2, 3, 1, 0).reshape(9 * 128, 32)
    dr1bm = dr1b.transpose(2, 3, 1, 0).reshape(9 * 32, 128)
    dtr = _call(_k_dec_trunk, B,
                (q, dw1m, db1, dr0am, dr0bm, dr1am, dr1bm),
                (True,) + (False,) * 6,
                jax.ShapeDtypeStruct((B, _L, 128), f32), True,
                scratch=(pltpu.VMEM((_L, 64), f32),
                         pltpu.VMEM((_L, 128), f32),
                         pltpu.VMEM((_L, 32), f32),
                         pltpu.VMEM((_M, 9 * 128), f32)))

    tw1m = jnp.stack(
        [jnp.concatenate([tw1[:, :, dy, dx]
                          for ao, dy in _T1MAP[r] for bo, dx in _T1MAP[s]],
                         axis=0)
         for r in (0, 1) for s in (0, 1)])
    w7 = jnp.zeros((4, 4, 64, 4, 4, 3), f32)
    for wr in (-1, 0, 1, 2):
        for p_r, u_r, dy in _T2GRP[wr]:
            for wc in (-1, 0, 1, 2):
                for p_c, u_c, dx in _T2GRP[wc]:
                    w7 = w7.at[wr + 1, wc + 1, :, 2 * p_r + u_r,
                               2 * p_c + u_c].set(tw2[:, :, dy, dx])
    w7 = w7.reshape(16 * 64, 48)
    b48 = jnp.tile(tb2, 16)
    xt = _call(_k_convt, B, (dtr, tw1m, tb1, w7, b48),
               (True,) + (False,) * 4,
               jax.ShapeDtypeStruct((B, _M, 48), f32), True,
               scratch=(pltpu.VMEM((4, _L, 64), f32),
                        pltpu.VMEM((_M, 16 * 64), f32)))

    xt = xt.reshape(B, 64, _W, 4, 4, 3)[:, :, :64]
    x_tilda = xt.transpose(0, 5, 1, 3, 2, 4).reshape(B, 3, 256, 256)
    return (loss, x_tilda, perplexity)
